# baseline (speedup 1.0000x reference)
import functools

import jax
import jax.numpy as jnp
from jax import lax
from jax.experimental import pallas as pl
from jax.experimental.pallas import tpu as pltpu
from jax.experimental.pallas import tpu_sc as plsc


@functools.lru_cache(maxsize=None)
def _build_gather(T: int, V: int, D: int):
    info = plsc.get_sparse_core_info()
    NC, NS = info.num_cores, info.num_subcores
    NW = NC * NS
    assert T % NW == 0, (T, NW)
    b_per_w = T // NW
    CHUNK = 32
    assert b_per_w % CHUNK == 0
    n_chunks = b_per_w // CHUNK

    mesh = plsc.VectorSubcoreMesh(core_axis_name="c", subcore_axis_name="s")

    @functools.partial(
        pl.kernel,
        mesh=mesh,
        out_type=jax.ShapeDtypeStruct((T, D), jnp.float32),
        scratch_types=[
            pltpu.VMEM((16,), jnp.int32),
        ],
    )
    def gather_kernel(table_hbm, idx_hbm, out_hbm, idx_v):
        wid = lax.axis_index("s") * NC + lax.axis_index("c")
        base = wid * b_per_w
        pltpu.sync_copy(idx_hbm.at[pl.ds(0, 16)], idx_v)
        pos0 = pl.multiple_of(idx_v[...][0], 8)
        pltpu.sync_copy(
            table_hbm.at[pl.ds(pos0 + base, b_per_w)],
            out_hbm.at[pl.ds(base, b_per_w)])

    return gather_kernel


def kernel(x, table, pos):
    T = x.shape[1]
    V, D = table.shape
    idx = jnp.arange(T, dtype=jnp.int32) + jnp.asarray(pos, dtype=jnp.int32)
    out = _build_gather(T, V, D)(table, idx)
    return out[None]


# speedup vs baseline: 23.7108x; 23.7108x over previous
import functools

import jax
import jax.numpy as jnp
from jax import lax
from jax.experimental import pallas as pl
from jax.experimental.pallas import tpu as pltpu
from jax.experimental.pallas import tpu_sc as plsc


@functools.lru_cache(maxsize=None)
def _build_gather(T: int, V: int, D: int):
    info = plsc.get_sparse_core_info()
    NC, NS = info.num_cores, info.num_subcores
    NW = NC * NS
    assert T % NW == 0, (T, NW)
    b_per_w = T // NW
    CHUNK = 32
    assert b_per_w % CHUNK == 0
    n_chunks = b_per_w // CHUNK

    mesh = plsc.VectorSubcoreMesh(core_axis_name="c", subcore_axis_name="s")

    @functools.partial(
        pl.kernel,
        mesh=mesh,
        out_type=jax.ShapeDtypeStruct((T, D), jnp.float32),
        scratch_types=[
            pltpu.VMEM((16,), jnp.int32),
            pltpu.VMEM((CHUNK, D), jnp.float32),
            pltpu.VMEM((CHUNK, D), jnp.float32),
            pltpu.SemaphoreType.DMA,
            pltpu.SemaphoreType.DMA,
            pltpu.SemaphoreType.DMA,
            pltpu.SemaphoreType.DMA,
        ],
    )
    def gather_kernel(table_hbm, idx_hbm, out_hbm, idx_v, buf0, buf1,
                      gsem0, gsem1, osem0, osem1):
        wid = lax.axis_index("s") * NC + lax.axis_index("c")
        base = wid * b_per_w
        pltpu.sync_copy(idx_hbm.at[pl.ds(0, 16)], idx_v)
        pos0 = pl.multiple_of(idx_v[...][0], 8)
        bufs = (buf0, buf1)
        gsems = (gsem0, gsem1)
        osems = (osem0, osem1)
        gather = [None, None]
        scatter = [None, None]
        gather[0] = pltpu.async_copy(
            table_hbm.at[pl.ds(pos0 + base, CHUNK)], bufs[0], gsems[0])
        for c in range(n_chunks):
            b = c & 1
            nb = b ^ 1
            if c + 1 < n_chunks:
                if scatter[nb] is not None:
                    scatter[nb].wait()
                gather[nb] = pltpu.async_copy(
                    table_hbm.at[pl.ds(pos0 + base + (c + 1) * CHUNK, CHUNK)],
                    bufs[nb], gsems[nb])
            gather[b].wait()
            scatter[b] = pltpu.async_copy(
                bufs[b], out_hbm.at[pl.ds(base + c * CHUNK, CHUNK)], osems[b])
        scatter[0].wait()
        scatter[1].wait()

    return gather_kernel


def kernel(x, table, pos):
    T = x.shape[1]
    V, D = table.shape
    idx = jnp.arange(T, dtype=jnp.int32) + jnp.asarray(pos, dtype=jnp.int32)
    out = _build_gather(T, V, D)(table, idx)
    return out[None]
